# baseline (speedup 1.0000x reference)
import functools
import math

import jax
import jax.numpy as jnp
from jax import lax
from jax.experimental import pallas as pl
from jax.experimental.pallas import tpu as pltpu
from jax.experimental.pallas import tpu_sc as plsc

D_MODEL = 2048
VOCAB = 100000
SCALE = math.sqrt(D_MODEL)

NC = 2
NS = 16
L = 16
NW = NC * NS

B_ROWS = 4
B_COLS = 4096
B_TOTAL = B_ROWS * B_COLS
B_PER_W = B_TOTAL // NW
C = 8
N_CHUNKS = B_PER_W // C
NBUF = 4
RING_ITERS = N_CHUNKS // NBUF
VECS = C * (D_MODEL // L)


def _rewrite_indices(xf):
    pos = jnp.arange(B_TOTAL, dtype=jnp.int32) % C
    prevv = jnp.roll(xf, 1)
    nextv = jnp.roll(xf, -1)
    eq_prev = (pos != 0) & (xf == prevv)
    eq_next = (pos != C - 1) & (xf == nextv)
    odd = (pos % 2) == 1
    modify = odd & (eq_prev | eq_next)

    c1 = (xf + 1) % VOCAB
    c2 = (xf + 2) % VOCAB
    c3 = (xf + 3) % VOCAB
    r = jnp.where(
        (c1 != prevv) & (c1 != nextv),
        c1,
        jnp.where((c2 != prevv) & (c2 != nextv), c2, c3),
    )
    new_idx = jnp.where(modify, r, xf)
    fix = jnp.where(
        modify, jnp.where(eq_prev, jnp.int32(-1), jnp.int32(1)), jnp.int32(0)
    )
    return new_idx, fix


def _sc_gather_scale(table, idx, fix):
    mesh = plsc.VectorSubcoreMesh(
        core_axis_name="c", subcore_axis_name="s", num_cores=NC, num_subcores=NS
    )

    @functools.partial(
        pl.kernel,
        out_type=jax.ShapeDtypeStruct((B_TOTAL, D_MODEL), jnp.float32),
        mesh=mesh,
        scratch_types=[
            pltpu.VMEM((B_PER_W,), jnp.int32),
            pltpu.VMEM((N_CHUNKS, 16), jnp.int32),
            [pltpu.VMEM((C, D_MODEL), jnp.float32) for _ in range(NBUF)],
            [pltpu.SemaphoreType.DMA for _ in range(NBUF)],
            [pltpu.SemaphoreType.DMA for _ in range(NBUF)],
        ],
    )
    def k(table_hbm, idx_hbm, fix_hbm, out_hbm, idx_v, fix_v, buf, gsem, ssem):
        wid = lax.axis_index("s") * NC + lax.axis_index("c")
        base = wid * B_PER_W
        pltpu.sync_copy(idx_hbm.at[wid], idx_v)
        pltpu.sync_copy(fix_hbm.at[wid], fix_v)

        def gather(j, b):
            pltpu.async_copy(
                table_hbm.at[idx_v.at[pl.ds(j * C, C)]], buf[b], gsem[b]
            )

        for b in range(NBUF - 1):
            gather(b, b)

        def step(j, b):
            pltpu.make_async_copy(
                table_hbm.at[idx_v.at[pl.ds(j * C, C)]], buf[b], gsem[b]
            ).wait()

            @plsc.parallel_loop(0, VECS, unroll=8)
            def _(i):
                r = lax.shift_right_logical(i, 7)
                col = pl.multiple_of(jnp.bitwise_and(i, 127) * L, L)
                sl = pl.ds(col, L)
                buf[b][r, sl] = buf[b][r, sl] * SCALE

            vfix = fix_v[j, pl.ds(0, 16)]
            for t in range(1, C, 2):
                f = vfix[t]

                @pl.when(f != 0)
                def _():
                    src = t + f

                    def cp(kk, cc):
                        sl = pl.ds(pl.multiple_of(kk * L, L), L)
                        buf[b][t, sl] = buf[b][src, sl]
                        return cc

                    lax.fori_loop(0, D_MODEL // L, cp, 0)

            pltpu.async_copy(buf[b], out_hbm.at[pl.ds(base + j * C, C)], ssem[b])

            bp = (b + 3) % NBUF
            @pl.when(j == 0)
            def _():
                gather(3, bp)

            @pl.when(jnp.logical_and(j >= 1, j + 3 < N_CHUNKS))
            def _():
                pltpu.make_async_copy(
                    buf[bp], out_hbm.at[pl.ds(base + (j - 1) * C, C)], ssem[bp]
                ).wait()
                gather(j + 3, bp)

        def outer(t, carry):
            for b in range(NBUF):
                step(t * NBUF + b, b)
            return carry

        lax.fori_loop(0, RING_ITERS, outer, 0)

        for q in range(NBUF):
            j = N_CHUNKS - NBUF + q
            pltpu.make_async_copy(
                buf[j % NBUF], out_hbm.at[pl.ds(base + j * C, C)], ssem[j % NBUF]
            ).wait()

    return k(table, idx, fix)


@jax.jit
def kernel(x, table):
    xf = x.reshape(B_TOTAL).astype(jnp.int32)
    new_idx, fix = _rewrite_indices(xf)
    fix_pad = jnp.zeros((NW, N_CHUNKS, 16), jnp.int32)
    fix_pad = fix_pad.at[:, :, :C].set(fix.reshape(NW, N_CHUNKS, C))
    out = _sc_gather_scale(table, new_idx.reshape(NW, B_PER_W), fix_pad)
    return out.reshape(x.shape[0], x.shape[1], D_MODEL)


# speedup vs baseline: 1.0357x; 1.0357x over previous
import functools
import math

import jax
import jax.numpy as jnp
from jax import lax
from jax.experimental import pallas as pl
from jax.experimental.pallas import tpu as pltpu
from jax.experimental.pallas import tpu_sc as plsc

D_MODEL = 2048
VOCAB = 100000
SCALE = math.sqrt(D_MODEL)

NC = 2
NS = 16
L = 16
NW = NC * NS

B_ROWS = 4
B_COLS = 4096
B_TOTAL = B_ROWS * B_COLS
B_PER_W = B_TOTAL // NW
C = 8
N_CHUNKS = B_PER_W // C
NBUF = 4
RING_ITERS = N_CHUNKS // NBUF
VECS = C * (D_MODEL // L)


def _rewrite_indices(xf):
    pos = jnp.arange(B_TOTAL, dtype=jnp.int32) % C
    prevv = jnp.roll(xf, 1)
    nextv = jnp.roll(xf, -1)
    eq_prev = (pos != 0) & (xf == prevv)
    eq_next = (pos != C - 1) & (xf == nextv)
    odd = (pos % 2) == 1
    modify = odd & (eq_prev | eq_next)

    c1 = (xf + 1) % VOCAB
    c2 = (xf + 2) % VOCAB
    c3 = (xf + 3) % VOCAB
    r = jnp.where(
        (c1 != prevv) & (c1 != nextv),
        c1,
        jnp.where((c2 != prevv) & (c2 != nextv), c2, c3),
    )
    new_idx = jnp.where(modify, r, xf)
    fix = jnp.where(
        modify, jnp.where(eq_prev, jnp.int32(-1), jnp.int32(1)), jnp.int32(0)
    )
    return new_idx, fix


def _sc_gather_scale(table, idx, fix):
    mesh = plsc.VectorSubcoreMesh(
        core_axis_name="c", subcore_axis_name="s", num_cores=NC, num_subcores=NS
    )

    @functools.partial(
        pl.kernel,
        out_type=jax.ShapeDtypeStruct((B_TOTAL, D_MODEL), jnp.float32),
        mesh=mesh,
        scratch_types=[
            pltpu.VMEM((B_PER_W,), jnp.int32),
            pltpu.VMEM((B_PER_W,), jnp.int32),
            [pltpu.VMEM((C, D_MODEL), jnp.float32) for _ in range(NBUF)],
            [pltpu.SemaphoreType.DMA for _ in range(NBUF)],
            [pltpu.SemaphoreType.DMA for _ in range(NBUF)],
        ],
    )
    def k(table_hbm, idx_hbm, fix_hbm, out_hbm, idx_v, fix_v, buf, gsem, ssem):
        wid = lax.axis_index("s") * NC + lax.axis_index("c")
        base = wid * B_PER_W
        pltpu.sync_copy(idx_hbm.at[wid], idx_v)
        pltpu.sync_copy(fix_hbm.at[wid], fix_v)

        def gather(j, b):
            pltpu.async_copy(
                table_hbm.at[idx_v.at[pl.ds(j * C, C)]], buf[b], gsem[b]
            )

        for b in range(NBUF - 1):
            gather(b, b)

        def step(t, j, b):
            pltpu.make_async_copy(
                table_hbm.at[idx_v.at[pl.ds(j * C, C)]], buf[b], gsem[b]
            ).wait()

            @plsc.parallel_loop(0, VECS, unroll=8)
            def _(i):
                r = lax.shift_right_logical(i, 7)
                col = pl.multiple_of(jnp.bitwise_and(i, 127) * L, L)
                sl = pl.ds(col, L)
                buf[b][r, sl] = buf[b][r, sl] * SCALE

            voff = pl.multiple_of((t * 2 + b // 2) * 16, 16)
            vfix = fix_v[pl.ds(voff, 16)]
            for tt in range(1, C, 2):
                f = vfix[(b % 2) * C + tt]

                @pl.when(f != 0)
                def _():
                    src = tt + f

                    def cp(kk, cc):
                        sl = pl.ds(pl.multiple_of(kk * L, L), L)
                        buf[b][tt, sl] = buf[b][src, sl]
                        return cc

                    lax.fori_loop(0, D_MODEL // L, cp, 0)

            pltpu.async_copy(buf[b], out_hbm.at[pl.ds(base + j * C, C)], ssem[b])

            bp = (b + 3) % NBUF
            @pl.when(j == 0)
            def _():
                gather(3, bp)

            @pl.when(jnp.logical_and(j >= 1, j + 3 < N_CHUNKS))
            def _():
                pltpu.make_async_copy(
                    buf[bp], out_hbm.at[pl.ds(base + (j - 1) * C, C)], ssem[bp]
                ).wait()
                gather(j + 3, bp)

        def outer(t, carry):
            for b in range(NBUF):
                step(t, t * NBUF + b, b)
            return carry

        lax.fori_loop(0, RING_ITERS, outer, 0)

        for q in range(NBUF):
            j = N_CHUNKS - NBUF + q
            pltpu.make_async_copy(
                buf[j % NBUF], out_hbm.at[pl.ds(base + j * C, C)], ssem[j % NBUF]
            ).wait()

    return k(table, idx, fix)


@jax.jit
def kernel(x, table):
    xf = x.reshape(B_TOTAL).astype(jnp.int32)
    new_idx, fix = _rewrite_indices(xf)
    out = _sc_gather_scale(
        table, new_idx.reshape(NW, B_PER_W), fix.reshape(NW, B_PER_W)
    )
    return out.reshape(x.shape[0], x.shape[1], D_MODEL)
